# Initial kernel scaffold; baseline (speedup 1.0000x reference)
import jax
import jax.numpy as jnp
from jax.experimental import pallas as pl


def kernel(x, edge_index, edge_attr, W_fc, W_attn, W_edge_att, W_e2n):
    raise NotImplementedError("write your pallas kernel here")


# speedup vs baseline: 2.6486x; 2.6486x over previous
import jax
import jax.numpy as jnp
from jax import lax
from jax.experimental import pallas as pl
from jax.experimental.pallas import tpu as pltpu
from jax.experimental.pallas import tpu_sc as plsc

N = 10000
E = 320000
D = 128
NC = 2
NS = 16
NW = NC * NS
EPW = E // NW
C = 80
ITERS = EPW // C


def _stage1_body(x_ref, wfc_ref, wattn_ref, wea_ref, eat_ref,
                 z_ref, als_ref, ald_ref, aex_ref):
    z = lax.dot_general(x_ref[...], wfc_ref[...], (((1,), (1,)), ((), ())))
    z_ref[...] = z
    als_ref[...] = lax.dot_general(z, wattn_ref[:, 0:D], (((1,), (1,)), ((), ())))
    ald_ref[...] = lax.dot_general(z, wattn_ref[:, D:2 * D], (((1,), (1,)), ((), ())))
    v4 = lax.dot_general(wattn_ref[:, 2 * D:2 * D + 4], wea_ref[...],
                         (((1,), (0,)), ((), ())))
    aex2 = lax.dot_general(v4, eat_ref[...], (((1,), (0,)), ((), ())))
    aex_ref[...] = aex2[0]


_stage1 = pl.pallas_call(
    _stage1_body,
    out_shape=(
        jax.ShapeDtypeStruct((N, D), jnp.float32),
        jax.ShapeDtypeStruct((N, 1), jnp.float32),
        jax.ShapeDtypeStruct((N, 1), jnp.float32),
        jax.ShapeDtypeStruct((E,), jnp.float32),
    ),
)


def _sc_body(als_hbm, ald_hbm, aex_hbm, src_hbm, dst_hbm,
             w_hbm,
             als_t, ald_t, sidx_v, didx_v, aexbuf, wbuf):
    cid = lax.axis_index("c")
    sid = lax.axis_index("s")
    wid = sid * NC + cid
    pltpu.sync_copy(als_hbm, als_t)
    pltpu.sync_copy(ald_hbm, ald_t)

    def chunk(it, carry):
        off = wid * EPW + it * C
        pltpu.sync_copy(src_hbm.at[pl.ds(off, C)], sidx_v)
        pltpu.sync_copy(dst_hbm.at[pl.ds(off, C)], didx_v)
        pltpu.sync_copy(aex_hbm.at[pl.ds(off, C)], aexbuf)
        for g in range(C // 16):
            sl = pl.ds(g * 16, 16)
            a_s = plsc.load_gather(als_t, [sidx_v[sl]])
            a_d = plsc.load_gather(ald_t, [didx_v[sl]])
            t = a_s + a_d + aexbuf[sl]
            t = jnp.where(t > 0.0, t, t * 0.01)
            wbuf[sl] = jnp.exp(t)
        pltpu.sync_copy(wbuf, w_hbm.at[pl.ds(off, C)])
        return carry

    lax.fori_loop(0, ITERS, chunk, 0)


_sc_attn = pl.kernel(
    _sc_body,
    out_type=jax.ShapeDtypeStruct((E,), jnp.float32),
    mesh=plsc.VectorSubcoreMesh(core_axis_name="c", subcore_axis_name="s"),
    compiler_params=pltpu.CompilerParams(needs_layout_passes=False),
    scratch_types=[
        pltpu.VMEM((N,), jnp.float32),
        pltpu.VMEM((N,), jnp.float32),
        pltpu.VMEM((C,), jnp.int32),
        pltpu.VMEM((C,), jnp.int32),
        pltpu.VMEM((C,), jnp.float32),
        pltpu.VMEM((C,), jnp.float32),
    ],
)


def _stage4_body(u_ref, s_ref, g0_ref, wea_ref, we2n_ref, out_ref):
    s = s_ref[...]
    m = lax.dot_general(wea_ref[...], we2n_ref[...], (((0,), (1,)), ((), ())))
    hv = u_ref[...] + lax.dot_general(g0_ref[...], m, (((1,), (0,)), ((), ())))
    out_ref[...] = hv / jnp.where(s == 0.0, 1.0, s)


_stage4 = pl.pallas_call(
    _stage4_body,
    out_shape=jax.ShapeDtypeStruct((N, D), jnp.float32),
)


def kernel(x, edge_index, edge_attr, W_fc, W_attn, W_edge_att, W_e2n):
    src = edge_index[0]
    dst = edge_index[1]
    eaT = edge_attr.T
    z, als, ald, aex = _stage1(x, W_fc, W_attn, W_edge_att, eaT)
    w = _sc_attn(als.reshape(N), ald.reshape(N), aex, src, dst)
    u = jax.ops.segment_sum(w[:, None] * z[src], dst, num_segments=N)
    s = jax.ops.segment_sum(w, dst, num_segments=N)
    g0 = jax.ops.segment_sum(w[:, None] * edge_attr, dst, num_segments=N)
    return _stage4(u, s[:, None], g0, W_edge_att, W_e2n)
